# Initial kernel scaffold; baseline (speedup 1.0000x reference)
import functools

import jax
import jax.numpy as jnp
from jax import lax
from jax.experimental import pallas as pl
from jax.experimental.pallas import tpu as pltpu
from jax.experimental.pallas import tpu_sc as plsc

N = 10000
E = 160000
NC = 2
NS = 16
LANES = 16
B = 128
K = -(-E // (NS * B))
EP = NS * K * B
RPT = N // NS
WB = 125
NZ = RPT // WB


def _mm1(x, W):
    n, kd = x.shape
    fh = W.shape[1] // 2
    bm = 1000
    r = n // bm

    def body(x_ref, w_ref, o_ref):
        o_ref[...] = jnp.dot(x_ref[...], w_ref[...],
                             preferred_element_type=jnp.float32)

    return pl.pallas_call(
        body,
        grid=(r, 2),
        in_specs=[pl.BlockSpec((bm, kd), lambda i, c: (i, 0)),
                  pl.BlockSpec((kd, fh), lambda i, c: (0, c))],
        out_specs=pl.BlockSpec((bm, fh), lambda i, c: (c * r + i, 0)),
        out_shape=jax.ShapeDtypeStruct((2 * n, fh), jnp.float32),
    )(x, W)


def _mm2(h_split, w2a, w2b):
    n2, kh = h_split.shape
    n = n2 // 2
    fh = w2a.shape[2]
    bm = 1000
    r = n // bm

    def body(ha_ref, hb_ref, wa_ref, wb_ref, o_ref):
        o_ref[...] = (
            jnp.dot(ha_ref[...], wa_ref[0], preferred_element_type=jnp.float32)
            + jnp.dot(hb_ref[...], wb_ref[0], preferred_element_type=jnp.float32))

    return pl.pallas_call(
        body,
        grid=(r, 2),
        in_specs=[pl.BlockSpec((bm, kh), lambda i, c: (i, 0)),
                  pl.BlockSpec((bm, kh), lambda i, c: (r + i, 0)),
                  pl.BlockSpec((1, kh, fh), lambda i, c: (c, 0, 0)),
                  pl.BlockSpec((1, kh, fh), lambda i, c: (c, 0, 0))],
        out_specs=pl.BlockSpec((bm, fh), lambda i, c: (c * r + i, 0)),
        out_shape=jax.ShapeDtypeStruct((2 * n, fh), jnp.float32),
    )(h_split, h_split, w2a, w2b)


def _make_agg(fh):
    nf = fh // LANES
    mesh = plsc.VectorSubcoreMesh(core_axis_name="c", subcore_axis_name="s",
                                  num_cores=NC, num_subcores=NS)

    @functools.partial(
        pl.kernel,
        out_type=jax.ShapeDtypeStruct((2 * N, fh), jnp.float32),
        mesh=mesh,
        scratch_types=[
            pltpu.VMEM_SHARED((N, fh), jnp.float32),
            pltpu.VMEM((K, B), jnp.int32),
            pltpu.VMEM((K, B), jnp.int32),
            pltpu.VMEM((K, B), jnp.float32),
            pltpu.VMEM((B, fh), jnp.float32),
            pltpu.VMEM((WB, fh), jnp.float32),
            pltpu.VMEM((fh,), jnp.float32),
            pltpu.SemaphoreType.DMA,
        ],
    )
    def agg(xw, srcq, dstq, wq, bias, out,
            acc, src_all, dst_all, w_all, rows, buf, bias_v, sem):
        c = lax.axis_index("c")
        s = lax.axis_index("s")
        cn = c * N

        pltpu.sync_copy(srcq.at[s], src_all)
        pltpu.sync_copy(dstq.at[s], dst_all)
        pltpu.sync_copy(wq.at[s], w_all)
        pltpu.sync_copy(bias.at[pl.ds(c * fh, fh)], bias_v)

        @pl.loop(0, K)
        def _adj(j):
            for q in range(B // LANES):
                sl = pl.ds(q * LANES, LANES)
                src_all[j, sl] = src_all[j, sl] + cn

        zero = jnp.zeros((LANES,), jnp.float32)

        @pl.loop(0, WB)
        def _zb(rr):
            for q in range(nf):
                buf[rr, pl.ds(q * LANES, LANES)] = zero

        for z in range(NZ):
            pltpu.sync_copy(buf, acc.at[pl.ds(s * RPT + z * WB, WB)])

        plsc.subcore_barrier()

        @pl.loop(0, K)
        def _step(j):
            pltpu.async_copy(xw.at[src_all.at[j]], rows, sem).wait()
            jb = jnp.full((LANES,), 0, jnp.int32) + j

            @pl.loop(0, B)
            def _scale(e):
                eb = jnp.full((LANES,), 0, jnp.int32) + e
                ws = plsc.load_gather(w_all, [jb, eb])
                for q in range(nf):
                    sl = pl.ds(q * LANES, LANES)
                    rows[e, sl] = rows[e, sl] * ws

            pltpu.sync_copy(rows, acc.at[dst_all.at[j]], add=True)

        plsc.subcore_barrier()

        for z in range(NZ):
            r0 = s * RPT + z * WB
            pltpu.sync_copy(acc.at[pl.ds(r0, WB)], buf)

            @pl.loop(0, WB)
            def _wb(rr):
                for q in range(nf):
                    sl = pl.ds(q * LANES, LANES)
                    buf[rr, sl] = jnp.maximum(buf[rr, sl] + bias_v[sl], 0.0)

            pltpu.sync_copy(buf, out.at[pl.ds(cn + r0, WB)])

    return agg


_agg128 = _make_agg(128)
_agg64 = _make_agg(64)


def kernel(x, edge_index, edge_weight, W1, b1, W2, b2):
    src = edge_index[0].astype(jnp.int32)
    dst = edge_index[1].astype(jnp.int32)
    w = edge_weight.astype(jnp.float32)
    pad = EP - E
    srcq = jnp.concatenate([src, jnp.zeros((pad,), jnp.int32)]).reshape(NS, K, B)
    dstq = jnp.concatenate([dst, jnp.zeros((pad,), jnp.int32)]).reshape(NS, K, B)
    wq = jnp.concatenate([w, jnp.zeros((pad,), jnp.float32)]).reshape(NS, K, B)

    kh = W2.shape[0] // 2
    fh2 = W2.shape[1] // 2
    w2q = W2.reshape(2, kh, 2, fh2).transpose(0, 2, 1, 3)

    xw1 = _mm1(x, W1)
    h = _agg128(xw1, srcq, dstq, wq, b1)
    xw2 = _mm2(h, w2q[0], w2q[1])
    o = _agg64(xw2, srcq, dstq, wq, b2)
    return jnp.concatenate([o[:N], o[N:]], axis=1)


# speedup vs baseline: 2.8739x; 2.8739x over previous
import functools

import jax
import jax.numpy as jnp
from jax import lax
from jax.experimental import pallas as pl
from jax.experimental.pallas import tpu as pltpu
from jax.experimental.pallas import tpu_sc as plsc

N = 10000
NP = 10240
E = 160000
NC = 2
NS = 16
LANES = 16
B = 128
K = -(-E // (NS * B))
EP = NS * K * B
RPT = NP // NS
WB = 64
NZ = RPT // WB
CH = 16


def _mm1(x, W):
    n, kd = x.shape
    fh = W.shape[1] // 2
    bm = 640
    r = n // bm

    def body(x_ref, w_ref, o_ref):
        o_ref[...] = jnp.dot(x_ref[...], w_ref[...],
                             preferred_element_type=jnp.float32)

    return pl.pallas_call(
        body,
        grid=(r, 2),
        in_specs=[pl.BlockSpec((bm, kd), lambda i, c: (i, 0)),
                  pl.BlockSpec((kd, fh), lambda i, c: (0, c))],
        out_specs=pl.BlockSpec((bm, fh), lambda i, c: (c * r + i, 0)),
        out_shape=jax.ShapeDtypeStruct((2 * n, fh), jnp.float32),
    )(x, W)


def _mm2(h_split, w2a, w2b):
    n2, kh = h_split.shape
    n = n2 // 2
    fh = w2a.shape[2]
    bm = 640
    r = n // bm

    def body(ha_ref, hb_ref, wa_ref, wb_ref, o_ref):
        o_ref[...] = (
            jnp.dot(ha_ref[...], wa_ref[0], preferred_element_type=jnp.float32)
            + jnp.dot(hb_ref[...], wb_ref[0], preferred_element_type=jnp.float32))

    return pl.pallas_call(
        body,
        grid=(r, 2),
        in_specs=[pl.BlockSpec((bm, kh), lambda i, c: (i, 0)),
                  pl.BlockSpec((bm, kh), lambda i, c: (r + i, 0)),
                  pl.BlockSpec((1, kh, fh), lambda i, c: (c, 0, 0)),
                  pl.BlockSpec((1, kh, fh), lambda i, c: (c, 0, 0))],
        out_specs=pl.BlockSpec((bm, fh), lambda i, c: (c * r + i, 0)),
        out_shape=jax.ShapeDtypeStruct((2 * n, fh), jnp.float32),
    )(h_split, h_split, w2a, w2b)


def _make_agg(fh):
    nf = fh // LANES
    mesh = plsc.VectorSubcoreMesh(core_axis_name="c", subcore_axis_name="s",
                                  num_cores=NC, num_subcores=NS)

    @functools.partial(
        pl.kernel,
        out_type=jax.ShapeDtypeStruct((2 * NP, fh), jnp.float32),
        mesh=mesh,
        compiler_params=pltpu.CompilerParams(use_tc_tiling_on_sc=False),
        scratch_types=[
            pltpu.VMEM_SHARED((NP, fh), jnp.float32),
            pltpu.VMEM((CH, 2, B), jnp.int32),
            pltpu.VMEM((CH * B + LANES,), jnp.float32),
            pltpu.VMEM((B, fh), jnp.float32),
            pltpu.VMEM((WB, fh), jnp.float32),
            pltpu.VMEM((fh,), jnp.float32),
            pltpu.SemaphoreType.DMA,
        ],
    )
    def agg(xw, idxq, wq, bias, out,
            acc, cb, wbuf, rows, buf, bias_v, sem):
        c = lax.axis_index("c")
        s = lax.axis_index("s")
        cn = c * NP

        pltpu.sync_copy(bias.at[pl.ds(c * fh, fh)], bias_v)

        zero = jnp.zeros((LANES,), jnp.float32)

        @pl.loop(0, WB)
        def _zb(rr):
            for q in range(nf):
                buf[rr, pl.ds(q * LANES, LANES)] = zero

        for z in range(NZ):
            pltpu.sync_copy(buf, acc.at[pl.ds(s * RPT + z * WB, WB)])

        plsc.subcore_barrier()

        for ch0 in range(0, K, CH):
            chn = min(CH, K - ch0)
            pltpu.sync_copy(idxq.at[s, pl.ds(ch0, chn)], cb.at[pl.ds(0, chn)])
            pltpu.sync_copy(wq.at[s, pl.ds(ch0 * B, chn * B)],
                            wbuf.at[pl.ds(0, chn * B)])

            @pl.loop(0, chn)
            def _adj(jj):
                for q in range(B // LANES):
                    sl = pl.ds(q * LANES, LANES)
                    cb[jj, 0, sl] = cb[jj, 0, sl] + cn

            @pl.loop(0, chn)
            def _step(jj):
                pltpu.async_copy(xw.at[cb.at[jj, 0]], rows, sem).wait()

                @pl.loop(0, B)
                def _scale(e):
                    wvec = wbuf[pl.ds(jj * B + e, LANES)]
                    ws = jnp.broadcast_to(wvec[0], (LANES,))
                    for q in range(nf):
                        sl = pl.ds(q * LANES, LANES)
                        rows[e, sl] = rows[e, sl] * ws

                pltpu.sync_copy(rows, acc.at[cb.at[jj, 1]], add=True)

        plsc.subcore_barrier()

        for z in range(NZ):
            r0 = s * RPT + z * WB
            pltpu.sync_copy(acc.at[pl.ds(r0, WB)], buf)

            @pl.loop(0, WB)
            def _wb(rr):
                for q in range(nf):
                    sl = pl.ds(q * LANES, LANES)
                    buf[rr, sl] = jnp.maximum(buf[rr, sl] + bias_v[sl], 0.0)

            pltpu.sync_copy(buf, out.at[pl.ds(cn + r0, WB)])

    return agg


_agg128 = _make_agg(128)
_agg64 = _make_agg(64)


def kernel(x, edge_index, edge_weight, W1, b1, W2, b2):
    src = edge_index[0].astype(jnp.int32)
    dst = edge_index[1].astype(jnp.int32)
    w = edge_weight.astype(jnp.float32)
    pad = EP - E
    srcq = jnp.concatenate([src, jnp.zeros((pad,), jnp.int32)]).reshape(NS, K, B)
    dstq = jnp.concatenate([dst, jnp.zeros((pad,), jnp.int32)]).reshape(NS, K, B)
    idxq = jnp.stack([srcq, dstq], axis=2)
    wq = jnp.concatenate([w, jnp.zeros((pad,), jnp.float32)]).reshape(NS, K * B)

    kh = W2.shape[0] // 2
    fh2 = W2.shape[1] // 2
    w2q = W2.reshape(2, kh, 2, fh2).transpose(0, 2, 1, 3)

    x_p = jnp.pad(x, ((0, NP - N), (0, 0)))

    xw1 = _mm1(x_p, W1)
    h = _agg128(xw1, idxq, wq, b1)
    xw2 = _mm2(h, w2q[0], w2q[1])
    o = _agg64(xw2, idxq, wq, b2)
    return jnp.concatenate([o[:N], o[NP:NP + N]], axis=1)
